# baseline (speedup 1.0000x reference)
import functools

import jax
import jax.numpy as jnp
import numpy as np
from jax import lax
from jax.experimental import pallas as pl
from jax.experimental.pallas import tpu as pltpu
from jax.experimental.pallas import tpu_sc as plsc

_FIELD_DIMS = [100000] * 26
_N_FIELDS = 26
_EMBED_DIM = 16
_D = _N_FIELDS * _EMBED_DIM
_B = 4096
_OFFS = np.concatenate(([0], np.cumsum(_FIELD_DIMS)[:-1])).astype(np.int32)
_BN_INV = float(1.0 / np.sqrt(1.0 + 1e-5))

_N_ROWS = _B * _N_FIELDS
_NW = 32
_RPW = _N_ROWS // _NW
_NG = _RPW // 16
_DEPTH = 16


def _sc_gather(table, idx2):
    mesh = plsc.VectorSubcoreMesh(core_axis_name="c", subcore_axis_name="s")

    @functools.partial(
        pl.kernel,
        mesh=mesh,
        out_type=jax.ShapeDtypeStruct((_NW, _RPW * _EMBED_DIM), jnp.float32),
        scratch_types=[
            pltpu.VMEM((_RPW,), jnp.int32),
            pltpu.VMEM((_RPW,), jnp.int32),
            pltpu.VMEM((_RPW,), jnp.int32),
            pltpu.VMEM((_DEPTH * 16, 1, _EMBED_DIM), jnp.float32),
            pltpu.VMEM((_RPW * _EMBED_DIM,), jnp.float32),
            pltpu.SemaphoreType.DMA,
        ],
        compiler_params=pltpu.CompilerParams(needs_layout_passes=False),
    )
    def k(table_hbm, idx_hbm, out_hbm, idx_v, sup_v, sub_v, stage_v, out_v,
          sem):
        wid = lax.axis_index("s") * 2 + lax.axis_index("c")
        pltpu.sync_copy(idx_hbm.at[wid], idx_v)

        def prep(g, _):
            v = idx_v[pl.ds(g * 16, 16)]
            sup_v[pl.ds(g * 16, 16)] = jnp.right_shift(v, 3)
            sub_v[pl.ds(g * 16, 16)] = jnp.bitwise_and(v, 7)
            return _
        lax.fori_loop(0, _NG, prep, 0)

        def fire_group(g):
            sups = sup_v[pl.ds(g * 16, 16)]
            subs = sub_v[pl.ds(g * 16, 16)]
            sbase = jnp.bitwise_and(g, _DEPTH - 1) * 16
            for l in range(16):
                pltpu.async_copy(
                    table_hbm.at[sups[l]].at[pl.ds(subs[l], 1)],
                    stage_v.at[sbase + l], sem)

        def drain16():
            for _ in range(16):
                pltpu.make_async_copy(
                    table_hbm.at[0].at[pl.ds(0, 1)], stage_v.at[0],
                    sem).wait()

        def subselect(g):
            slots = jnp.bitwise_and(g, _DEPTH - 1) * 16 + lax.iota(
                jnp.int32, 16)
            zeros = jnp.zeros((16,), jnp.int32)
            obase = (g * 16 + lax.iota(jnp.int32, 16)) * _EMBED_DIM
            for e in range(_EMBED_DIM):
                vals = plsc.load_gather(
                    stage_v, [slots, zeros, jnp.full((16,), e, jnp.int32)])
                plsc.store_scatter(out_v, [obase + e], vals)

        for g in range(_DEPTH):
            fire_group(g)

        def body(g, _):
            drain16()
            subselect(g)
            fire_group(g + _DEPTH)
            return _
        lax.fori_loop(0, _NG - _DEPTH, body, 0)

        def tail(g, _):
            drain16()
            subselect(g)
            return _
        lax.fori_loop(_NG - _DEPTH, _NG, tail, 0)

        pltpu.sync_copy(out_v, out_hbm.at[wid])

    return k(table, idx2)


def _dense_body(emb_ref, w0_ref, b0_ref, g0_ref, be0_ref, w1_ref, b1_ref,
                g1_ref, be1_ref, cw_ref, cb_ref, lw_ref, lb_ref, out_ref):
    emb = emb_ref[...]
    xl = emb
    for i in range(3):
        w = cw_ref[i, :]
        xw = jnp.sum(xl * w[None, :], axis=1, keepdims=True)
        xl = emb * xw + cb_ref[i, :][None, :] + xl
    h = jnp.dot(emb, w0_ref[...], preferred_element_type=jnp.float32)
    h = (h + b0_ref[...]) * (g0_ref[...] * _BN_INV) + be0_ref[...]
    h = jnp.maximum(h, 0.0)
    h = jnp.dot(h, w1_ref[...], preferred_element_type=jnp.float32)
    h = (h + b1_ref[...]) * (g1_ref[...] * _BN_INV) + be1_ref[...]
    h = jnp.maximum(h, 0.0)
    y = jnp.dot(xl, lw_ref[:_D, :], preferred_element_type=jnp.float32)
    y = y + jnp.dot(h, lw_ref[_D:, :], preferred_element_type=jnp.float32)
    y = y + lb_ref[...]
    out_ref[...] = jax.nn.sigmoid(y)


def _tc_dense(emb, w0, b0, g0, be0, w1, b1, g1, be1, cw, cb, lw, lb):
    blk = 512
    grid = _B // blk
    f0 = w0.shape[1]
    f1 = w1.shape[1]
    const = lambda i: (0, 0)
    out = pl.pallas_call(
        _dense_body,
        grid=(grid,),
        in_specs=[
            pl.BlockSpec((blk, _D), lambda i: (i, 0)),
            pl.BlockSpec((_D, f0), const),
            pl.BlockSpec((1, f0), const),
            pl.BlockSpec((1, f0), const),
            pl.BlockSpec((1, f0), const),
            pl.BlockSpec((f0, f1), const),
            pl.BlockSpec((1, f1), const),
            pl.BlockSpec((1, f1), const),
            pl.BlockSpec((1, f1), const),
            pl.BlockSpec((3, _D), const),
            pl.BlockSpec((3, _D), const),
            pl.BlockSpec((_D + f1, 1), const),
            pl.BlockSpec((1, 1), const),
        ],
        out_specs=pl.BlockSpec((blk, 1), lambda i: (i, 0)),
        out_shape=jax.ShapeDtypeStruct((_B, 1), jnp.float32),
    )(emb, w0, b0.reshape(1, f0), g0.reshape(1, f0), be0.reshape(1, f0),
      w1, b1.reshape(1, f1), g1.reshape(1, f1), be1.reshape(1, f1),
      cw, cb, lw, lb.reshape(1, 1))
    return out.reshape(_B)


def kernel(x, table, mlp_W0, mlp_b0, mlp_g0, mlp_be0, mlp_W1, mlp_b1,
           mlp_g1, mlp_be1, cross_w, cross_b, lin_W, lin_b):
    idx = (x + jnp.asarray(_OFFS)[None, :]).reshape(_NW, _RPW)
    one = lin_b[0] * 0.0 + 1.0
    rows = _sc_gather(table.reshape(-1, 8, _EMBED_DIM) * one, idx)
    emb = rows.reshape(_B, _D)
    return _tc_dense(emb, mlp_W0, mlp_b0, mlp_g0, mlp_be0, mlp_W1, mlp_b1,
                     mlp_g1, mlp_be1, cross_w, cross_b, lin_W, lin_b)


# speedup vs baseline: 1.2251x; 1.2251x over previous
import functools

import jax
import jax.numpy as jnp
import numpy as np
from jax import lax
from jax.experimental import pallas as pl
from jax.experimental.pallas import tpu as pltpu
from jax.experimental.pallas import tpu_sc as plsc

_FIELD_DIMS = [100000] * 26
_N_FIELDS = 26
_EMBED_DIM = 16
_D = _N_FIELDS * _EMBED_DIM
_B = 4096
_OFFS = np.concatenate(([0], np.cumsum(_FIELD_DIMS)[:-1])).astype(np.int32)
_BN_INV = float(1.0 / np.sqrt(1.0 + 1e-5))

_N_ROWS = _B * _N_FIELDS
_NW = 32
_RPW = _N_ROWS // _NW
_NG = _RPW // 16
_DEPTH = 16


def _sc_gather(table, idx2):
    mesh = plsc.VectorSubcoreMesh(core_axis_name="c", subcore_axis_name="s")

    @functools.partial(
        pl.kernel,
        mesh=mesh,
        out_type=jax.ShapeDtypeStruct((_NW, _RPW * _EMBED_DIM), jnp.float32),
        scratch_types=[
            pltpu.VMEM((_RPW,), jnp.int32),
            pltpu.VMEM((_RPW,), jnp.int32),
            pltpu.VMEM((_RPW,), jnp.int32),
            pltpu.VMEM((_DEPTH * 16, 1, _EMBED_DIM), jnp.float32),
            pltpu.VMEM((_RPW * _EMBED_DIM,), jnp.float32),
            pltpu.SemaphoreType.DMA,
        ],
        compiler_params=pltpu.CompilerParams(needs_layout_passes=False),
    )
    def k(table_hbm, idx_hbm, out_hbm, idx_v, sup_v, sub_v, stage_v, out_v,
          sem):
        wid = lax.axis_index("s") * 2 + lax.axis_index("c")
        pltpu.sync_copy(idx_hbm.at[wid], idx_v)

        def prep(g, _):
            v = idx_v[pl.ds(g * 16, 16)]
            sup_v[pl.ds(g * 16, 16)] = jnp.right_shift(v, 3)
            sub_v[pl.ds(g * 16, 16)] = jnp.bitwise_and(v, 7)
            return _
        lax.fori_loop(0, _NG, prep, 0)

        def fire_group(g):
            sups = sup_v[pl.ds(g * 16, 16)]
            subs = sub_v[pl.ds(g * 16, 16)]
            sbase = jnp.bitwise_and(g, _DEPTH - 1) * 16
            for l in range(16):
                pltpu.async_copy(
                    table_hbm.at[sups[l]].at[pl.ds(subs[l], 1)],
                    stage_v.at[sbase + l], sem)

        def drain16():
            for _ in range(16):
                pltpu.make_async_copy(
                    table_hbm.at[0].at[pl.ds(0, 1)], stage_v.at[0],
                    sem).wait()

        def subselect(g):
            slots = jnp.bitwise_and(g, _DEPTH - 1) * 16 + lax.iota(
                jnp.int32, 16)
            zeros = jnp.zeros((16,), jnp.int32)
            obase = (g * 16 + lax.iota(jnp.int32, 16)) * _EMBED_DIM
            for e in range(_EMBED_DIM):
                vals = plsc.load_gather(
                    stage_v, [slots, zeros, jnp.full((16,), e, jnp.int32)])
                plsc.store_scatter(out_v, [obase + e], vals)

        for g in range(_DEPTH):
            fire_group(g)

        def body(g, _):
            drain16()
            subselect(g)
            fire_group(g + _DEPTH)
            return _
        lax.fori_loop(0, _NG - _DEPTH, body, 0)

        def tail(g, _):
            drain16()
            subselect(g)
            return _
        lax.fori_loop(_NG - _DEPTH, _NG, tail, 0)

        pltpu.sync_copy(out_v, out_hbm.at[wid])

    return k(table, idx2)


def _dense_body(emb_ref, w0_ref, b0_ref, g0_ref, be0_ref, w1_ref, b1_ref,
                g1_ref, be1_ref, cw_ref, cb_ref, lw_ref, lb_ref, out_ref):
    emb = emb_ref[...]
    xl = emb
    for i in range(3):
        w = cw_ref[i, :]
        xw = jnp.sum(xl * w[None, :], axis=1, keepdims=True)
        xl = emb * xw + cb_ref[i, :][None, :] + xl
    h = jnp.dot(emb, w0_ref[...], preferred_element_type=jnp.float32)
    h = (h + b0_ref[...]) * (g0_ref[...] * _BN_INV) + be0_ref[...]
    h = jnp.maximum(h, 0.0)
    h = jnp.dot(h, w1_ref[...], preferred_element_type=jnp.float32)
    h = (h + b1_ref[...]) * (g1_ref[...] * _BN_INV) + be1_ref[...]
    h = jnp.maximum(h, 0.0)
    y = jnp.dot(xl, lw_ref[:_D, :], preferred_element_type=jnp.float32)
    y = y + jnp.dot(h, lw_ref[_D:, :], preferred_element_type=jnp.float32)
    y = y + lb_ref[...]
    out_ref[...] = jax.nn.sigmoid(y)


def _tc_dense(emb, w0, b0, g0, be0, w1, b1, g1, be1, cw, cb, lw, lb):
    blk = 512
    grid = _B // blk
    f0 = w0.shape[1]
    f1 = w1.shape[1]
    const = lambda i: (0, 0)
    out = pl.pallas_call(
        _dense_body,
        grid=(grid,),
        in_specs=[
            pl.BlockSpec((blk, _D), lambda i: (i, 0)),
            pl.BlockSpec((_D, f0), const),
            pl.BlockSpec((1, f0), const),
            pl.BlockSpec((1, f0), const),
            pl.BlockSpec((1, f0), const),
            pl.BlockSpec((f0, f1), const),
            pl.BlockSpec((1, f1), const),
            pl.BlockSpec((1, f1), const),
            pl.BlockSpec((1, f1), const),
            pl.BlockSpec((3, _D), const),
            pl.BlockSpec((3, _D), const),
            pl.BlockSpec((_D + f1, 1), const),
            pl.BlockSpec((1, 1), const),
        ],
        out_specs=pl.BlockSpec((blk, 1), lambda i: (i, 0)),
        out_shape=jax.ShapeDtypeStruct((_B, 1), jnp.float32),
    )(emb, w0, b0.reshape(1, f0), g0.reshape(1, f0), be0.reshape(1, f0),
      w1, b1.reshape(1, f1), g1.reshape(1, f1), be1.reshape(1, f1),
      cw, cb, lw, lb.reshape(1, 1))
    return out.reshape(_B)


def kernel(x, table, mlp_W0, mlp_b0, mlp_g0, mlp_be0, mlp_W1, mlp_b1,
           mlp_g1, mlp_be1, cross_w, cross_b, lin_W, lin_b):
    idx = (x + jnp.asarray(_OFFS)[None, :]).reshape(_NW, _RPW)
    rows = _sc_gather(table.reshape(-1, 8, _EMBED_DIM), idx)
    emb = rows.reshape(_B, _D)
    return _tc_dense(emb, mlp_W0, mlp_b0, mlp_g0, mlp_be0, mlp_W1, mlp_b1,
                     mlp_g1, mlp_be1, cross_w, cross_b, lin_W, lin_b)
